# Initial kernel scaffold; baseline (speedup 1.0000x reference)
import jax
import jax.numpy as jnp
from jax.experimental import pallas as pl


def kernel(yhat, target):
    raise NotImplementedError("write your pallas kernel here")


# speedup vs baseline: 1.8084x; 1.8084x over previous
import functools
import math

import jax
import jax.numpy as jnp
from jax.experimental import pallas as pl

_VOCAB = 100000
_PAD = 0
_SMOOTH = 0.1
_CONF = 1.0 - _SMOOTH
_FILL = _SMOOTH / (_VOCAB - 2)
_C = _CONF * math.log(_CONF) + _SMOOTH * math.log(_FILL)

_BLOCK_COLS = 2048


def _ls_kernel(y_ref, t_ref, out_ref, *, block_cols, vocab):
    k = pl.program_id(0)
    base = k * block_cols
    col = base + jax.lax.broadcasted_iota(jnp.int32, (1, block_cols), 1)
    x = jnp.where(col < vocab, y_ref[...], 0.0)

    t = t_ref[...]
    valid = (t != _PAD).astype(jnp.float32)

    s_part = jnp.sum(x, axis=1, keepdims=True)
    s_valid = jnp.sum(s_part * valid, keepdims=True)

    g = jnp.where(col == t, x, 0.0)
    g_sum = jnp.sum(jnp.sum(g, axis=1, keepdims=True) * valid, keepdims=True)

    contrib = -_FILL * s_valid - (_CONF - _FILL) * g_sum

    @pl.when(k == 0)
    def _():
        z_sum = jnp.sum(x[:, 0:1] * valid, keepdims=True)
        n_valid = jnp.sum(valid, keepdims=True)
        out_ref[...] = contrib + _FILL * z_sum + n_valid * _C

    @pl.when(k != 0)
    def _():
        out_ref[...] += contrib


def kernel(yhat, target):
    n, vocab = yhat.shape
    t2 = target.astype(jnp.int32).reshape(n, 1)
    grid = pl.cdiv(vocab, _BLOCK_COLS)
    out = pl.pallas_call(
        functools.partial(_ls_kernel, block_cols=_BLOCK_COLS, vocab=vocab),
        grid=(grid,),
        in_specs=[
            pl.BlockSpec((n, _BLOCK_COLS), lambda k: (0, k)),
            pl.BlockSpec((n, 1), lambda k: (0, 0)),
        ],
        out_specs=pl.BlockSpec((1, 1), lambda k: (0, 0)),
        out_shape=jax.ShapeDtypeStruct((1, 1), jnp.float32),
    )(yhat, t2)
    return out[0, 0]
